# Initial kernel scaffold; baseline (speedup 1.0000x reference)
import jax
import jax.numpy as jnp
from jax.experimental import pallas as pl


def kernel(token_ids, weights):
    raise NotImplementedError("write your pallas kernel here")


# speedup vs baseline: 1.0949x; 1.0949x over previous
import functools

import jax
import jax.numpy as jnp
from jax import lax
from jax.experimental import pallas as pl
from jax.experimental.pallas import tpu as pltpu
from jax.experimental.pallas import tpu_sc as plsc

_D = 32
_C = 1024


@functools.lru_cache(maxsize=None)
def _make_gather(B, V):
    info = plsc.get_sparse_core_info()
    NC, NS = info.num_cores, info.num_subcores
    NW = NC * NS
    assert B % NW == 0
    b_per_w = B // NW
    assert b_per_w % _C == 0
    n_chunks = b_per_w // _C

    mesh = plsc.VectorSubcoreMesh(core_axis_name="c", subcore_axis_name="s")

    @functools.partial(
        pl.kernel,
        out_type=jax.ShapeDtypeStruct((B, _D), jnp.float32),
        mesh=mesh,
        scratch_types=[
            pltpu.VMEM((_C,), jnp.int32),
            pltpu.VMEM((_C, _D), jnp.float32),
            pltpu.SemaphoreType.DMA,
        ],
        compiler_params=pltpu.CompilerParams(use_tc_tiling_on_sc=False),
    )
    def gather_kernel(table_hbm, idx_hbm, out_hbm, idx_v, rows_v, sem):
        wid = lax.axis_index("s") * NC + lax.axis_index("c")
        base = wid * b_per_w

        @pl.loop(0, n_chunks)
        def _chunk(g):
            off = base + g * _C
            pltpu.sync_copy(idx_hbm.at[pl.ds(off, _C)], idx_v)
            pltpu.async_copy(table_hbm.at[idx_v], rows_v, sem).wait()
            pltpu.sync_copy(rows_v, out_hbm.at[pl.ds(off, _C)])

    return gather_kernel


def kernel(token_ids, weights):
    B0, S = token_ids.shape
    B = B0 * S
    idx = token_ids.reshape(B).astype(jnp.int32)
    out = _make_gather(B, weights.shape[0])(weights, idx)
    return out.reshape(B0, S, _D)


# speedup vs baseline: 1.1104x; 1.0142x over previous
import functools

import jax
import jax.numpy as jnp
from jax import lax
from jax.experimental import pallas as pl
from jax.experimental.pallas import tpu as pltpu
from jax.experimental.pallas import tpu_sc as plsc

_D = 32
_C = 640
_NB = 4


@functools.lru_cache(maxsize=None)
def _make_gather(B, V):
    info = plsc.get_sparse_core_info()
    NC, NS = info.num_cores, info.num_subcores
    NW = NC * NS
    assert B % NW == 0
    b_per_w = B // NW
    assert b_per_w % (_C * _NB) == 0
    n_outer = b_per_w // (_C * _NB)

    mesh = plsc.VectorSubcoreMesh(core_axis_name="c", subcore_axis_name="s")

    @functools.partial(
        pl.kernel,
        out_type=jax.ShapeDtypeStruct((B, _D), jnp.float32),
        mesh=mesh,
        scratch_types=[
            pltpu.VMEM((b_per_w,), jnp.int32),
            [pltpu.VMEM((_C, _D), jnp.float32) for _ in range(_NB)],
            [pltpu.SemaphoreType.DMA for _ in range(_NB)],
            [pltpu.SemaphoreType.DMA for _ in range(_NB)],
        ],
        compiler_params=pltpu.CompilerParams(use_tc_tiling_on_sc=False),
    )
    def gather_kernel(table_hbm, idx_hbm, out_hbm, idx_v, rows_v, sem_g, sem_w):
        wid = lax.axis_index("s") * NC + lax.axis_index("c")
        base = wid * b_per_w

        pltpu.sync_copy(idx_hbm.at[pl.ds(base, b_per_w)], idx_v)

        def issue_gather(g, b):
            pltpu.async_copy(
                table_hbm.at[idx_v.at[pl.ds(g * _C, _C)]], rows_v[b], sem_g[b]
            )

        def issue_wb(g, b):
            pltpu.async_copy(
                rows_v[b], out_hbm.at[pl.ds(base + g * _C, _C)], sem_w[b]
            )

        def wait_gather(b):
            pltpu.make_async_copy(
                out_hbm.at[pl.ds(base, _C)], rows_v[b], sem_g[b]
            ).wait()

        def wait_wb(b):
            pltpu.make_async_copy(
                rows_v[b], out_hbm.at[pl.ds(base, _C)], sem_w[b]
            ).wait()

        for b in range(_NB):
            issue_gather(b, b)

        @pl.loop(0, n_outer - 1)
        def _outer(o):
            for b in range(_NB):
                wait_gather(b)
                issue_wb(o * _NB + b, b)
            for b in range(_NB):
                wait_wb(b)
                issue_gather((o + 1) * _NB + b, b)

        o_last = n_outer - 1
        for b in range(_NB):
            wait_gather(b)
            issue_wb(o_last * _NB + b, b)
        for b in range(_NB):
            wait_wb(b)

    return gather_kernel


def kernel(token_ids, weights):
    B0, S = token_ids.shape
    B = B0 * S
    idx = token_ids.reshape(B).astype(jnp.int32)
    out = _make_gather(B, weights.shape[0])(weights, idx)
    return out.reshape(B0, S, _D)


# speedup vs baseline: 1.8118x; 1.6316x over previous
import functools

import jax
import jax.numpy as jnp
from jax import lax
from jax.experimental import pallas as pl
from jax.experimental.pallas import tpu as pltpu
from jax.experimental.pallas import tpu_sc as plsc

_D = 32
_C = 1024
_NB = 2


@functools.lru_cache(maxsize=None)
def _make_gather(B, S, V):
    info = plsc.get_sparse_core_info()
    NC, NS = info.num_cores, info.num_subcores
    NW = NC * NS
    n_bblk = B // _C
    n_chunks = S * n_bblk
    assert n_chunks % NW == 0
    c_per_w = n_chunks // NW
    assert c_per_w >= _NB + 1 and (c_per_w - 1) % _NB == 0

    mesh = plsc.VectorSubcoreMesh(core_axis_name="c", subcore_axis_name="s")

    @functools.partial(
        pl.kernel,
        out_type=jax.ShapeDtypeStruct((B, S, _D), jnp.float32),
        mesh=mesh,
        scratch_types=[
            [pltpu.VMEM((_C,), jnp.int32) for _ in range(_NB)],
            [pltpu.VMEM((_C, _D), jnp.float32) for _ in range(_NB)],
            [pltpu.SemaphoreType.DMA for _ in range(_NB)],
            [pltpu.SemaphoreType.DMA for _ in range(_NB)],
        ],
        compiler_params=pltpu.CompilerParams(use_tc_tiling_on_sc=False),
    )
    def gather_kernel(table_hbm, idx_hbm, out_hbm, idx_v, rows_v, sem_g, sem_w):
        wid = lax.axis_index("s") * NC + lax.axis_index("c")

        def chunk_coords(j):
            c = j * NW + wid
            return c // n_bblk, (c % n_bblk) * _C

        def load_idx_and_gather(j, b):
            s, b0 = chunk_coords(j)
            pltpu.sync_copy(idx_hbm.at[s, pl.ds(b0, _C)], idx_v[b])
            pltpu.async_copy(table_hbm.at[idx_v[b]], rows_v[b], sem_g[b])

        def issue_wb(j, b):
            s, b0 = chunk_coords(j)
            pltpu.async_copy(
                rows_v[b], out_hbm.at[pl.ds(b0, _C), s], sem_w[b]
            )

        def wait_gather(b):
            pltpu.make_async_copy(
                table_hbm.at[pl.ds(0, _C)], rows_v[b], sem_g[b]
            ).wait()

        def wait_wb(b):
            pltpu.make_async_copy(
                rows_v[b], table_hbm.at[pl.ds(0, _C)], sem_w[b]
            ).wait()

        for b in range(_NB):
            load_idx_and_gather(b, b)

        @pl.loop(0, (c_per_w - 1) // _NB - 1)
        def _outer(o):
            for b in range(_NB):
                j = o * _NB + b
                wait_gather(b)
                issue_wb(j, b)
            for b in range(_NB):
                wait_wb(b)
                load_idx_and_gather(o * _NB + b + _NB, b)

        j0 = c_per_w - 1 - _NB
        for b in range(_NB):
            wait_gather(b)
            issue_wb(j0 + b, b)
        wait_wb(0)
        load_idx_and_gather(c_per_w - 1, 0)
        wait_wb(1)
        wait_gather(0)
        issue_wb(c_per_w - 1, 0)
        wait_wb(0)

    return gather_kernel


def kernel(token_ids, weights):
    B0, S = token_ids.shape
    idx_t = token_ids.T.astype(jnp.int32)
    return _make_gather(B0, S, weights.shape[0])(weights, idx_t)
